# baseline (speedup 1.0000x reference)
import dataclasses
import functools
import jax
import jax.numpy as jnp
from jax import lax
from jax.experimental import pallas as pl
from jax.experimental.pallas import tpu as pltpu
from jax.experimental.pallas import tpu_sc as plsc

N = 10000
E = 160000
EPS = 1e-5

NC = 2
NS = 16
NW = NC * NS
RPT = 320
NPAD = NW * RPT + 16
SLAB = 2000
NSLAB = E // SLAB

_mesh = plsc.VectorSubcoreMesh(core_axis_name="c", subcore_axis_name="s")

_sc_params = pltpu.CompilerParams()
if "needs_layout_passes" in pltpu.CompilerParams.__dataclass_fields__:
    _sc_params = dataclasses.replace(_sc_params, needs_layout_passes=False)


def _edge_body(k_hbm, q_hbm, v_hbm, e_hbm, src_hbm, dst_hbm,
               agg_hbm,
               dslab_v, sslab_v, stgd_v, stgs_v, stge_v,
               kbuf_v, qbuf_v, vbuf_v, ebuf_v, agg_v, sem, sem2):
    b = lax.axis_index("s") * NC + lax.axis_index("c")
    row0 = b * RPT

    def zrow(r, _):
        agg_v[pl.ds(r * 16, 16)] = jnp.zeros((16,), jnp.float32)
        return 0
    lax.fori_loop(0, (RPT + 1) * 16, zrow, 0)

    lanes = lax.iota(jnp.int32, 16)

    def slab(s, _):
        pltpu.async_copy(dst_hbm.at[pl.ds(s * SLAB, SLAB)], dslab_v, sem).wait()
        pltpu.async_copy(src_hbm.at[pl.ds(s * SLAB, SLAB)], sslab_v, sem).wait()

        def grp(g, cnt):
            sl = pl.ds(g * 16, 16)
            d16 = dslab_v[sl]
            s16 = sslab_v[sl]
            m = (d16 >= row0) & (d16 < row0 + RPT)
            mi = m.astype(jnp.int32)
            inc = plsc.cumsum(mi)
            pos16 = jnp.where(m, cnt + inc - mi, SLAB + 16)
            plsc.store_scatter(stgd_v, [pos16], d16)
            plsc.store_scatter(stgs_v, [pos16], s16)
            eid16 = s * SLAB + g * 16 + lanes
            plsc.store_scatter(stge_v, [pos16], eid16)
            npop = plsc.all_reduce_population_count(m)
            return cnt + npop[0]
        cnt = lax.fori_loop(0, SLAB // 16, grp, 0)

        tail = pl.ds(cnt, 16)
        stgd_v[tail] = jnp.full((16,), row0 + RPT, jnp.int32)
        stgs_v[tail] = jnp.zeros((16,), jnp.int32)
        stge_v[tail] = jnp.zeros((16,), jnp.int32)

        def chunk(mi, _):
            isl = pl.ds(mi * 16, 16)
            c1 = pltpu.async_copy(k_hbm.at[stgd_v.at[isl]], kbuf_v, sem)
            c2 = pltpu.async_copy(q_hbm.at[stgs_v.at[isl]], qbuf_v, sem)
            c3 = pltpu.async_copy(v_hbm.at[stgs_v.at[isl]], vbuf_v, sem)
            c4 = pltpu.async_copy(e_hbm.at[stge_v.at[isl]], ebuf_v, sem2)
            c1.wait(); c2.wait(); c3.wait(); c4.wait()
            d16 = stgd_v[isl]
            for j in range(16):
                ld = d16[j] - row0
                for c in range(16):
                    fsl = pl.ds(c * 16, 16)
                    den = 1.0 + jnp.exp(-(kbuf_v[j, fsl] + qbuf_v[j, fsl]
                                          + ebuf_v[j, fsl]))
                    idx16 = ld * 256 + c * 16 + lanes
                    plsc.addupdate_scatter(agg_v, [idx16],
                                           vbuf_v[j, fsl] / den)
            return 0
        lax.fori_loop(0, (cnt + 15) // 16, chunk, 0)
        return 0
    lax.fori_loop(0, NSLAB, slab, 0)

    pltpu.async_copy(agg_v.at[pl.ds(0, RPT * 256)],
                     agg_hbm.at[pl.ds(row0 * 256, RPT * 256)], sem).wait()


def _k_edge(k, q, v, e, src, dst):
    f = pl.kernel(
        _edge_body,
        out_type=jax.ShapeDtypeStruct((NW * RPT * 256,), jnp.float32),
        mesh=_mesh,
        scratch_types=[pltpu.VMEM((SLAB,), jnp.int32),
                       pltpu.VMEM((SLAB,), jnp.int32),
                       pltpu.VMEM((SLAB + 32,), jnp.int32),
                       pltpu.VMEM((SLAB + 32,), jnp.int32),
                       pltpu.VMEM((SLAB + 32,), jnp.int32),
                       pltpu.VMEM((16, 256), jnp.float32),
                       pltpu.VMEM((16, 256), jnp.float32),
                       pltpu.VMEM((16, 256), jnp.float32),
                       pltpu.VMEM((16, 256), jnp.float32),
                       pltpu.VMEM(((RPT + 1) * 256,), jnp.float32),
                       pltpu.SemaphoreType.DMA,
                       pltpu.SemaphoreType.DMA],
        compiler_params=_sc_params,
    )
    return f(k, q, v, e, src, dst)


def _h_kernel(combine_ref, wn_ref, bn_ref, h_ref):
    h_ref[...] = (jnp.dot(combine_ref[...], wn_ref[...],
                          preferred_element_type=jnp.float32)
                  + bn_ref[...])


def _e32_kernel(ea_ref, we_ref, be_ref, e32_ref):
    e32_ref[...] = (jnp.dot(ea_ref[...], we_ref[...],
                            preferred_element_type=jnp.float32)
                    + be_ref[...])


def _pre(combine_out, p, edge_attr):
    h = pl.pallas_call(
        _h_kernel,
        out_shape=jax.ShapeDtypeStruct((N, 256), jnp.float32),
    )(combine_out, p['lin_node_W'], p['lin_node_b'][None])
    ea = jnp.pad(edge_attr, ((0, 0), (0, 6)))
    wep = jnp.pad(p['lin_edge_W'], ((0, 6), (0, 0)))
    grid = 16
    blk = E // grid
    e32 = pl.pallas_call(
        _e32_kernel,
        grid=(grid,),
        in_specs=[pl.BlockSpec((blk, 32), lambda i: (i, 0)),
                  pl.BlockSpec((32, 32), lambda i: (0, 0)),
                  pl.BlockSpec((1, 32), lambda i: (0, 0))],
        out_specs=pl.BlockSpec((blk, 32), lambda i: (i, 0)),
        out_shape=jax.ShapeDtypeStruct((E, 32), jnp.float32),
    )(ea, wep, p['lin_edge_b'][None])
    return h, e32


def _kqv_kernel(x_ref, wk_ref, wq_ref, wv_ref, ws_ref, bk_ref, bq_ref,
                bv_ref, bs_ref, k_ref, q_ref, v_ref, s_ref):
    x = x_ref[...]
    k_ref[...] = jnp.dot(x, wk_ref[...], preferred_element_type=jnp.float32) + bk_ref[...]
    q_ref[...] = jnp.dot(x, wq_ref[...], preferred_element_type=jnp.float32) + bq_ref[...]
    v_ref[...] = jnp.dot(x, wv_ref[...], preferred_element_type=jnp.float32) + bv_ref[...]
    s_ref[...] = jnp.dot(x, ws_ref[...], preferred_element_type=jnp.float32) + bs_ref[...]


def _kqv(xpad, p, li):
    shp = jax.ShapeDtypeStruct((NPAD, 256), jnp.float32)
    return pl.pallas_call(
        _kqv_kernel,
        out_shape=(shp, shp, shp, shp),
    )(xpad, p['Wk'][li], p['Wq'][li], p['Wv'][li], p['Ws'][li],
      p['bk'][li][None], p['bq'][li][None], p['bv'][li][None],
      (p['bias'][li])[None])


def _edge_lin_kernel(e32_ref, we_ref, be_ref, e_ref):
    e_ref[...] = (jnp.dot(e32_ref[...], we_ref[...],
                          preferred_element_type=jnp.float32) + be_ref[...])


def _edge_lin(e32, p, li):
    grid = 16
    blk = E // grid
    return pl.pallas_call(
        _edge_lin_kernel,
        grid=(grid,),
        in_specs=[pl.BlockSpec((blk, 32), lambda i: (i, 0)),
                  pl.BlockSpec((32, 256), lambda i: (0, 0)),
                  pl.BlockSpec((1, 256), lambda i: (0, 0))],
        out_specs=pl.BlockSpec((blk, 256), lambda i: (i, 0)),
        out_shape=jax.ShapeDtypeStruct((E, 256), jnp.float32),
    )(e32, p['We'][li], p['be'][li][None])


def _post_kernel(agg_ref, skip_ref, sc_ref, bi_ref, out_ref):
    h = agg_ref[...] + skip_ref[...]
    mu = h.mean(-1, keepdims=True)
    var = ((h - mu) ** 2).mean(-1, keepdims=True)
    out_ref[...] = (h - mu) / jnp.sqrt(var + EPS) * sc_ref[...] + bi_ref[...]


def _post(agg, skip, p, li):
    return pl.pallas_call(
        _post_kernel,
        out_shape=jax.ShapeDtypeStruct((N, 256), jnp.float32),
    )(agg, skip, p['ln_scale'][li][None], p['ln_bias'][li][None])


def _final_kernel(h0_ref, h1_ref, h2_ref, w0_ref, w1_ref, w2_ref, bh_ref,
                  aw1_ref, ab1_ref, aw2_ref, ab2_ref, fw_ref, fb_ref,
                  out_ref):
    f = (jnp.dot(h0_ref[...], w0_ref[...], preferred_element_type=jnp.float32)
         + jnp.dot(h1_ref[...], w1_ref[...], preferred_element_type=jnp.float32)
         + jnp.dot(h2_ref[...], w2_ref[...], preferred_element_type=jnp.float32)
         + bh_ref[...])
    t = jnp.tanh(jnp.dot(f, aw1_ref[...], preferred_element_type=jnp.float32)
                 + ab1_ref[...])
    s = jnp.dot(t, aw2_ref[...], preferred_element_type=jnp.float32) + ab2_ref[0, 0]
    s = s[:, 0]
    m = jnp.max(s)
    w = jnp.exp(s - m)
    w = w / jnp.sum(w)
    pooled = jnp.sum(f * w[:, None], axis=0)
    out_ref[...] = jax.nn.sigmoid(
        jnp.sum(pooled * fw_ref[:, 0]) + fb_ref[0, 0])[None, None]


def _final(h0, h1, h2, p):
    w = p['lin_hidden_W']
    out = pl.pallas_call(
        _final_kernel,
        out_shape=jax.ShapeDtypeStruct((1, 1), jnp.float32),
    )(h0, h1, h2, w[0:256], w[256:512], w[512:768],
      p['lin_hidden_b'][None], p['attn_W1'], p['attn_b1'][None],
      p['attn_W2'], p['attn_b2'][None], p['final_W'], p['final_b'][None])
    return out[0, 0]


def kernel(combine_out, edge_attr, params, edge_index):
    p = params
    src = edge_index[0]
    dst = edge_index[1]
    h, e32 = _pre(combine_out, p, edge_attr)
    hiddens = []
    cur = h
    for li in range(3):
        xpad = jnp.pad(cur, ((0, NPAD - N), (0, 0)))
        k, q, v, skip = _kqv(xpad, p, li)
        e = _edge_lin(e32, p, li)
        agg = _k_edge(k, q, v, e, src, dst).reshape(NW * RPT, 256)
        cur = _post(agg[:N], skip[:N], p, li)
        hiddens.append(cur)
    return _final(hiddens[0], hiddens[1], hiddens[2], p)


# speedup vs baseline: 1.6380x; 1.6380x over previous
import dataclasses
import functools
import jax
import jax.numpy as jnp
from jax import lax
from jax.experimental import pallas as pl
from jax.experimental.pallas import tpu as pltpu
from jax.experimental.pallas import tpu_sc as plsc

N = 10000
E = 160000
EPS = 1e-5

NC = 2
NS = 16
NW = NC * NS
RPT = 320
NPAD = NW * RPT + 16
SLAB = 800
NSLAB = E // SLAB
CAPT = 3072
STG = CAPT + 80

_mesh = plsc.VectorSubcoreMesh(core_axis_name="c", subcore_axis_name="s")

_sc_params = pltpu.CompilerParams()
if "needs_layout_passes" in pltpu.CompilerParams.__dataclass_fields__:
    _sc_params = dataclasses.replace(_sc_params, needs_layout_passes=False)


def _edge_body(k_hbm, q_hbm, v_hbm, e_hbm, src_hbm, dst_hbm,
               agg_hbm,
               dslab_a, dslab_b, sslab_a, sslab_b,
               stgd_v, stgs_v, stge_v,
               kb0, qb0, vb0, eb0, kb1, qb1, vb1, eb1,
               agg_v, sem_a, sem_b, sem_c, sem_d):
    b = lax.axis_index("s") * NC + lax.axis_index("c")
    row0 = b * RPT
    lanes = lax.iota(jnp.int32, 16)

    def zrow(r, _):
        agg_v[pl.ds(r * 16, 16)] = jnp.zeros((16,), jnp.float32)
        return 0
    lax.fori_loop(0, (RPT + 1) * 16, zrow, 0)

    def issue_slab(sp, dbuf, sbuf, sem):
        pltpu.async_copy(dst_hbm.at[pl.ds(sp * SLAB, SLAB)], dbuf, sem)
        pltpu.async_copy(src_hbm.at[pl.ds(sp * SLAB, SLAB)], sbuf, sem)

    def wait_slab(dbuf, sbuf, sem):
        pltpu.make_async_copy(dst_hbm.at[pl.ds(0, SLAB)], dbuf, sem).wait()
        pltpu.make_async_copy(src_hbm.at[pl.ds(0, SLAB)], sbuf, sem).wait()

    def compact(sp, dbuf, sbuf, cnt):
        def grp(g, cnt):
            sl = pl.ds(g * 16, 16)
            d16 = dbuf[sl]
            s16 = sbuf[sl]
            m = (d16 >= row0) & (d16 < row0 + RPT)
            mi = m.astype(jnp.int32)
            inc = plsc.cumsum(mi)
            pos16 = jnp.where(m, cnt + inc - mi, CAPT + 64)
            plsc.store_scatter(stgd_v, [pos16], d16)
            plsc.store_scatter(stgs_v, [pos16], s16)
            eid16 = sp * SLAB + g * 16 + lanes
            plsc.store_scatter(stge_v, [pos16], eid16)
            npop = plsc.all_reduce_population_count(m)
            return cnt + npop[0]
        return lax.fori_loop(0, SLAB // 16, grp, cnt)

    def issue_chunk(ch, kb, qb, vb, eb, sem):
        isl = pl.ds(ch * 16, 16)
        pltpu.async_copy(k_hbm.at[stgd_v.at[isl]], kb, sem)
        pltpu.async_copy(q_hbm.at[stgs_v.at[isl]], qb, sem)
        pltpu.async_copy(v_hbm.at[stgs_v.at[isl]], vb, sem)
        pltpu.async_copy(e_hbm.at[stge_v.at[isl]], eb, sem)

    def wait_chunk(kb, qb, vb, eb, sem):
        isl = pl.ds(0, 16)
        pltpu.make_async_copy(k_hbm.at[isl], kb, sem).wait()
        pltpu.make_async_copy(q_hbm.at[isl], qb, sem).wait()
        pltpu.make_async_copy(v_hbm.at[isl], vb, sem).wait()
        pltpu.make_async_copy(e_hbm.at[isl], eb, sem).wait()

    def compute_chunk(ch, kb, qb, vb, eb):
        d16 = stgd_v[pl.ds(ch * 16, 16)]
        for j in range(16):
            ld = d16[j] - row0

            def cgrp(c, _):
                fsl = pl.ds(c * 16, 16)
                den = 1.0 + jnp.exp(-(kb[j, fsl] + qb[j, fsl] + eb[j, fsl]))
                idx16 = ld * 256 + c * 16 + lanes
                plsc.addupdate_scatter(agg_v, [idx16], vb[j, fsl] / den)
                return 0
            lax.fori_loop(0, 16, cgrp, 0, unroll=4)

    def batch_body(state):
        sp0 = state

        def pfill(i, _):
            sl = pl.ds(i * 16, 16)
            stgd_v[sl] = jnp.full((16,), row0 + RPT, jnp.int32)
            stgs_v[sl] = jnp.zeros((16,), jnp.int32)
            stge_v[sl] = jnp.zeros((16,), jnp.int32)
            return 0
        lax.fori_loop(0, STG // 16, pfill, 0)

        issue_slab(sp0, dslab_a, sslab_a, sem_c)

        def scan_cond(st):
            sp, cnt = st
            return (sp < NSLAB) & (cnt <= CAPT - 2 * SLAB)

        def scan_body(st):
            sp, cnt = st
            issue_slab(sp + 1, dslab_b, sslab_b, sem_d)
            wait_slab(dslab_a, sslab_a, sem_c)
            cnt = compact(sp, dslab_a, sslab_a, cnt)
            issue_slab(lax.min(sp + 2, NSLAB - 2), dslab_a, sslab_a, sem_c)
            wait_slab(dslab_b, sslab_b, sem_d)
            cnt = compact(sp + 1, dslab_b, sslab_b, cnt)
            return sp + 2, cnt

        sp, cnt = lax.while_loop(scan_cond, scan_body, (sp0, 0))
        wait_slab(dslab_a, sslab_a, sem_c)

        npairs = (cnt + 31) // 32
        issue_chunk(0, kb0, qb0, vb0, eb0, sem_a)

        def pair(i, _):
            issue_chunk(2 * i + 1, kb1, qb1, vb1, eb1, sem_b)
            wait_chunk(kb0, qb0, vb0, eb0, sem_a)
            compute_chunk(2 * i, kb0, qb0, vb0, eb0)
            issue_chunk(2 * i + 2, kb0, qb0, vb0, eb0, sem_a)
            wait_chunk(kb1, qb1, vb1, eb1, sem_b)
            compute_chunk(2 * i + 1, kb1, qb1, vb1, eb1)
            return 0
        lax.fori_loop(0, npairs, pair, 0)
        wait_chunk(kb0, qb0, vb0, eb0, sem_a)
        return sp

    lax.while_loop(lambda sp: sp < NSLAB, batch_body, 0)

    pltpu.async_copy(agg_v.at[pl.ds(0, RPT * 256)],
                     agg_hbm.at[pl.ds(row0 * 256, RPT * 256)], sem_a).wait()


def _k_edge(k, q, v, e, src, dst):
    f = pl.kernel(
        _edge_body,
        out_type=jax.ShapeDtypeStruct((NW * RPT * 256,), jnp.float32),
        mesh=_mesh,
        scratch_types=[pltpu.VMEM((SLAB,), jnp.int32),
                       pltpu.VMEM((SLAB,), jnp.int32),
                       pltpu.VMEM((SLAB,), jnp.int32),
                       pltpu.VMEM((SLAB,), jnp.int32),
                       pltpu.VMEM((STG,), jnp.int32),
                       pltpu.VMEM((STG,), jnp.int32),
                       pltpu.VMEM((STG,), jnp.int32),
                       pltpu.VMEM((16, 256), jnp.float32),
                       pltpu.VMEM((16, 256), jnp.float32),
                       pltpu.VMEM((16, 256), jnp.float32),
                       pltpu.VMEM((16, 256), jnp.float32),
                       pltpu.VMEM((16, 256), jnp.float32),
                       pltpu.VMEM((16, 256), jnp.float32),
                       pltpu.VMEM((16, 256), jnp.float32),
                       pltpu.VMEM((16, 256), jnp.float32),
                       pltpu.VMEM(((RPT + 1) * 256,), jnp.float32),
                       pltpu.SemaphoreType.DMA,
                       pltpu.SemaphoreType.DMA,
                       pltpu.SemaphoreType.DMA,
                       pltpu.SemaphoreType.DMA],
        compiler_params=_sc_params,
    )
    return f(k, q, v, e, src, dst)


def _h_kernel(combine_ref, wn_ref, bn_ref, h_ref):
    h_ref[...] = (jnp.dot(combine_ref[...], wn_ref[...],
                          preferred_element_type=jnp.float32)
                  + bn_ref[...])


def _e32_kernel(ea_ref, we_ref, be_ref, e32_ref):
    e32_ref[...] = (jnp.dot(ea_ref[...], we_ref[...],
                            preferred_element_type=jnp.float32)
                    + be_ref[...])


def _pre(combine_out, p, edge_attr):
    h = pl.pallas_call(
        _h_kernel,
        out_shape=jax.ShapeDtypeStruct((N, 256), jnp.float32),
    )(combine_out, p['lin_node_W'], p['lin_node_b'][None])
    ea = jnp.pad(edge_attr, ((0, 0), (0, 6)))
    wep = jnp.pad(p['lin_edge_W'], ((0, 6), (0, 0)))
    grid = 16
    blk = E // grid
    e32 = pl.pallas_call(
        _e32_kernel,
        grid=(grid,),
        in_specs=[pl.BlockSpec((blk, 32), lambda i: (i, 0)),
                  pl.BlockSpec((32, 32), lambda i: (0, 0)),
                  pl.BlockSpec((1, 32), lambda i: (0, 0))],
        out_specs=pl.BlockSpec((blk, 32), lambda i: (i, 0)),
        out_shape=jax.ShapeDtypeStruct((E, 32), jnp.float32),
    )(ea, wep, p['lin_edge_b'][None])
    return h, e32


def _kqv_kernel(x_ref, wk_ref, wq_ref, wv_ref, ws_ref, bk_ref, bq_ref,
                bv_ref, bs_ref, k_ref, q_ref, v_ref, s_ref):
    x = x_ref[...]
    k_ref[...] = jnp.dot(x, wk_ref[...], preferred_element_type=jnp.float32) + bk_ref[...]
    q_ref[...] = jnp.dot(x, wq_ref[...], preferred_element_type=jnp.float32) + bq_ref[...]
    v_ref[...] = jnp.dot(x, wv_ref[...], preferred_element_type=jnp.float32) + bv_ref[...]
    s_ref[...] = jnp.dot(x, ws_ref[...], preferred_element_type=jnp.float32) + bs_ref[...]


def _kqv(xpad, p, li):
    shp = jax.ShapeDtypeStruct((NPAD, 256), jnp.float32)
    return pl.pallas_call(
        _kqv_kernel,
        out_shape=(shp, shp, shp, shp),
    )(xpad, p['Wk'][li], p['Wq'][li], p['Wv'][li], p['Ws'][li],
      p['bk'][li][None], p['bq'][li][None], p['bv'][li][None],
      (p['bias'][li])[None])


def _edge_lin_kernel(e32_ref, we_ref, be_ref, e_ref):
    e_ref[...] = (jnp.dot(e32_ref[...], we_ref[...],
                          preferred_element_type=jnp.float32) + be_ref[...])


def _edge_lin(e32, p, li):
    grid = 16
    blk = E // grid
    return pl.pallas_call(
        _edge_lin_kernel,
        grid=(grid,),
        in_specs=[pl.BlockSpec((blk, 32), lambda i: (i, 0)),
                  pl.BlockSpec((32, 256), lambda i: (0, 0)),
                  pl.BlockSpec((1, 256), lambda i: (0, 0))],
        out_specs=pl.BlockSpec((blk, 256), lambda i: (i, 0)),
        out_shape=jax.ShapeDtypeStruct((E, 256), jnp.float32),
    )(e32, p['We'][li], p['be'][li][None])


def _post_kernel(agg_ref, skip_ref, sc_ref, bi_ref, out_ref):
    h = agg_ref[...] + skip_ref[...]
    mu = h.mean(-1, keepdims=True)
    var = ((h - mu) ** 2).mean(-1, keepdims=True)
    out_ref[...] = (h - mu) / jnp.sqrt(var + EPS) * sc_ref[...] + bi_ref[...]


def _post(agg, skip, p, li):
    return pl.pallas_call(
        _post_kernel,
        out_shape=jax.ShapeDtypeStruct((N, 256), jnp.float32),
    )(agg, skip, p['ln_scale'][li][None], p['ln_bias'][li][None])


def _final_kernel(h0_ref, h1_ref, h2_ref, w0_ref, w1_ref, w2_ref, bh_ref,
                  aw1_ref, ab1_ref, aw2_ref, ab2_ref, fw_ref, fb_ref,
                  out_ref):
    f = (jnp.dot(h0_ref[...], w0_ref[...], preferred_element_type=jnp.float32)
         + jnp.dot(h1_ref[...], w1_ref[...], preferred_element_type=jnp.float32)
         + jnp.dot(h2_ref[...], w2_ref[...], preferred_element_type=jnp.float32)
         + bh_ref[...])
    t = jnp.tanh(jnp.dot(f, aw1_ref[...], preferred_element_type=jnp.float32)
                 + ab1_ref[...])
    s = jnp.dot(t, aw2_ref[...], preferred_element_type=jnp.float32) + ab2_ref[0, 0]
    s = s[:, 0]
    m = jnp.max(s)
    w = jnp.exp(s - m)
    w = w / jnp.sum(w)
    pooled = jnp.sum(f * w[:, None], axis=0)
    out_ref[...] = jax.nn.sigmoid(
        jnp.sum(pooled * fw_ref[:, 0]) + fb_ref[0, 0])[None, None]


def _final(h0, h1, h2, p):
    w = p['lin_hidden_W']
    out = pl.pallas_call(
        _final_kernel,
        out_shape=jax.ShapeDtypeStruct((1, 1), jnp.float32),
    )(h0, h1, h2, w[0:256], w[256:512], w[512:768],
      p['lin_hidden_b'][None], p['attn_W1'], p['attn_b1'][None],
      p['attn_W2'], p['attn_b2'][None], p['final_W'], p['final_b'][None])
    return out[0, 0]


def kernel(combine_out, edge_attr, params, edge_index):
    p = params
    src = edge_index[0]
    dst = edge_index[1]
    h, e32 = _pre(combine_out, p, edge_attr)
    hiddens = []
    cur = h
    for li in range(3):
        xpad = jnp.pad(cur, ((0, NPAD - N), (0, 0)))
        k, q, v, skip = _kqv(xpad, p, li)
        e = _edge_lin(e32, p, li)
        agg = _k_edge(k, q, v, e, src, dst).reshape(NW * RPT, 256)
        cur = _post(agg[:N], skip[:N], p, li)
        hiddens.append(cur)
    return _final(hiddens[0], hiddens[1], hiddens[2], p)
